# baseline (speedup 1.0000x reference)
import functools

import jax
import jax.numpy as jnp
from jax import lax
from jax.experimental import pallas as pl
from jax.experimental.pallas import tpu as pltpu
from jax.experimental.pallas import tpu_sc as plsc

_K = 81
_N = 4096
_NV = _N // 16
_INT_MIN = -2147483648
_W = 4
_WSZ = _W * _N
_NW = 32
_R_BOT = _N - _K + 1
_SC_ROWS = 2560
_BR = 256


def _row_select(xw, ow, hist, candk, candi, roff, lane, lane32, ones16,
                zero16):
    for i in range(32):
        hist[pl.ds(i * 16, 16)] = zero16

    @plsc.parallel_loop(0, _NV, unroll=8)
    def _(j):
        b = lax.bitcast_convert_type(xw[pl.ds(roff + j * 16, 16)], jnp.int32)
        u = b ^ ((b >> 31) | jnp.int32(_INT_MIN))
        digit = lax.shift_right_logical(u, 27)
        plsc.addupdate_scatter(hist, [lane32 + digit], ones16)

    acc0 = zero16
    acc1 = zero16
    for l in range(16):
        acc0 = acc0 + hist[pl.ds(l * 32, 16)]
        acc1 = acc1 + hist[pl.ds(l * 32 + 16, 16)]
    p0 = plsc.cumsum(acc0)
    p1 = plsc.cumsum(acc1) + p0[15]
    m0 = p0 < _R_BOT
    m1 = p1 < _R_BOT
    bstar = (plsc.all_reduce_population_count(m0)[0]
             + plsc.all_reduce_population_count(m1)[0])
    pb_low = jnp.sum(jnp.where(m0, acc0, 0)) + jnp.sum(jnp.where(m1, acc1, 0))
    inb0 = jnp.logical_and(p0 >= _R_BOT, (p0 - acc0) < _R_BOT)
    inb1 = jnp.logical_and(p1 >= _R_BOT, (p1 - acc1) < _R_BOT)
    n_c = jnp.sum(jnp.where(inb0, acc0, 0)) + jnp.sum(jnp.where(inb1, acc1, 0))
    c_above = jnp.int32(_N) - pb_low - n_c
    m_rank = jnp.int32(_K) - c_above

    @plsc.parallel_loop(0, _NV, unroll=4, carry=zero16)
    def cnt(j, c):
        v = xw[pl.ds(roff + j * 16, 16)]
        b = lax.bitcast_convert_type(v, jnp.int32)
        u = b ^ ((b >> 31) | jnp.int32(_INT_MIN))
        digit = lax.shift_right_logical(u, 27)
        ow[pl.ds(roff + j * 16, 16)] = jnp.where(digit > bstar, v,
                                                 jnp.float32(0.0))
        inb = digit == bstar
        slot = lax.shift_left(c, 4) | lane
        plsc.store_scatter(candk, [slot], u ^ jnp.int32(_INT_MIN), mask=inb)
        plsc.store_scatter(candi, [slot], j * 16 + lane, mask=inb)
        return c + jnp.where(inb, 1, 0)

    mx = jnp.max(cnt)

    t = zero16 + ((bstar << 27) ^ jnp.int32(_INT_MIN))
    for shift in range(26, -1, -1):
        cand_t = t + jnp.int32(1 << shift)

        @plsc.parallel_loop(0, mx, carry=zero16)
        def acc_ge(s, acc):
            kv = candk[pl.ds(s * 16, 16)]
            ge = jnp.logical_and(kv >= cand_t, cnt > s)
            return acc + plsc.all_reduce_population_count(ge)

        t = jnp.where(acc_ge >= m_rank, cand_t, t)

    @plsc.parallel_loop(0, mx)
    def _(s):
        kv = candk[pl.ds(s * 16, 16)]
        iv = candi[pl.ds(s * 16, 16)]
        ge = jnp.logical_and(kv >= t, cnt > s)
        vf = lax.bitcast_convert_type(
            kv ^ ((kv >> 31) & jnp.int32(0x7FFFFFFF)), jnp.float32)
        plsc.store_scatter(ow, [roff + iv], vf, mask=ge)


def _sc_wta(rpw, row_off, x_hbm, out_hbm, xw0, xw1, ow0, ow1, hist, candk,
            candi, si0, si1, so0, so1):
    cid = lax.axis_index("c")
    sid = lax.axis_index("s")
    wid = sid * 2 + cid
    lane = lax.iota(jnp.int32, 16)
    lane32 = lane * 32
    ones16 = jnp.ones((16,), jnp.int32)
    zero16 = jnp.zeros((16,), jnp.int32)
    xwb = [xw0, xw1]
    owb = [ow0, ow1]
    sin = [si0, si1]
    sout = [so0, so1]
    nwin = rpw // _W
    base_out = wid * rpw * _N
    base_in = (row_off + wid * rpw) * _N

    for b in range(2):
        pltpu.make_async_copy(x_hbm.at[pl.ds(base_in + b * _WSZ, _WSZ)],
                              xwb[b], sin[b]).start()

    def pair(g, c):
        for b in range(2):
            w = g * 2 + b
            gin = base_in + w * _WSZ
            gout = base_out + w * _WSZ
            pltpu.make_async_copy(x_hbm.at[pl.ds(gin, _WSZ)], xwb[b],
                                  sin[b]).wait()

            @pl.when(g > 0)
            def _():
                pltpu.make_async_copy(owb[b],
                                      out_hbm.at[pl.ds(gout - 2 * _WSZ,
                                                       _WSZ)],
                                      sout[b]).wait()

            def row(r, c2):
                _row_select(xwb[b], owb[b], hist, candk, candi, r * _N,
                            lane, lane32, ones16, zero16)
                return c2

            lax.fori_loop(0, _W, row, jnp.int32(0))
            pltpu.make_async_copy(owb[b], out_hbm.at[pl.ds(gout, _WSZ)],
                                  sout[b]).start()

            @pl.when(w + 2 < nwin)
            def _():
                pltpu.make_async_copy(x_hbm.at[pl.ds(gin + 2 * _WSZ, _WSZ)],
                                      xwb[b], sin[b]).start()
        return c

    lax.fori_loop(0, nwin // 2, pair, jnp.int32(0))

    for b in range(2):
        pltpu.make_async_copy(owb[b], out_hbm.at[pl.ds(base_out, _WSZ)],
                              sout[b]).wait()


def _wta_tc_body(x_ref, o_ref):
    xb = x_ref[...]
    b = lax.bitcast_convert_type(xb, jnp.int32)
    skey = b ^ ((b >> 31) & jnp.int32(0x7FFFFFFF))
    rows = xb.shape[0]
    t = jnp.full((rows, 1), jnp.int32(_INT_MIN))
    steps = [jnp.int32(_INT_MIN)] + [jnp.int32(1 << s)
                                     for s in range(30, -1, -1)]
    for step in steps:
        cand = t + step
        cnt = jnp.sum((skey >= cand).astype(jnp.int32), axis=1, keepdims=True)
        t = jnp.where(cnt >= _K, cand, t)
    o_ref[...] = jnp.where(skey >= t, xb, jnp.float32(0.0))


def kernel(x):
    B, S, N = x.shape
    rows = B * S
    xf2 = x.reshape(rows, N)
    rows_sc = _SC_ROWS if rows % 256 == 0 and rows > _SC_ROWS else rows
    rows_tc = rows - rows_sc

    out_tc = None
    if rows_tc:
        out_tc = pl.pallas_call(
            _wta_tc_body,
            grid=(rows_tc // _BR,),
            in_specs=[pl.BlockSpec((_BR, N), lambda i: (i, 0))],
            out_specs=pl.BlockSpec((_BR, N), lambda i: (i, 0)),
            out_shape=jax.ShapeDtypeStruct((rows, N), jnp.float32),
            compiler_params=pltpu.CompilerParams(
                dimension_semantics=("parallel",),
            ),
        )(xf2)

    rpw = rows_sc // _NW
    mesh = plsc.VectorSubcoreMesh(core_axis_name="c", subcore_axis_name="s")
    run = functools.partial(
        pl.kernel,
        mesh=mesh,
        out_type=jax.ShapeDtypeStruct((rows_sc * N,), jnp.float32),
        scratch_types=[
            pltpu.VMEM((_WSZ,), jnp.float32),
            pltpu.VMEM((_WSZ,), jnp.float32),
            pltpu.VMEM((_WSZ,), jnp.float32),
            pltpu.VMEM((_WSZ,), jnp.float32),
            pltpu.VMEM((512,), jnp.int32),
            pltpu.VMEM((_N + 16,), jnp.int32),
            pltpu.VMEM((_N + 16,), jnp.int32),
            pltpu.SemaphoreType.DMA,
            pltpu.SemaphoreType.DMA,
            pltpu.SemaphoreType.DMA,
            pltpu.SemaphoreType.DMA,
        ],
        compiler_params=pltpu.CompilerParams(needs_layout_passes=False),
    )(functools.partial(_sc_wta, rpw, rows_tc))
    out_sc = run(xf2.reshape(-1)).reshape(rows_sc, N)

    if out_tc is None:
        return out_sc.reshape(B, S, N)

    def _patch_body(full_ref, sc_ref, o_ref):
        o_ref[...] = sc_ref[...]

    blk_off = rows_tc // _BR
    out = pl.pallas_call(
        _patch_body,
        grid=(rows_sc // _BR,),
        in_specs=[
            pl.BlockSpec((_BR, N), lambda i: (i + blk_off, 0)),
            pl.BlockSpec((_BR, N), lambda i: (i, 0)),
        ],
        out_specs=pl.BlockSpec((_BR, N), lambda i: (i + blk_off, 0)),
        out_shape=jax.ShapeDtypeStruct((rows, N), jnp.float32),
        input_output_aliases={0: 0},
        compiler_params=pltpu.CompilerParams(
            dimension_semantics=("parallel",),
        ),
    )(out_tc, out_sc)
    return out.reshape(B, S, N)


# speedup vs baseline: 1.1149x; 1.1149x over previous
import functools

import jax
import jax.numpy as jnp
from jax import lax
from jax.experimental import pallas as pl
from jax.experimental.pallas import tpu as pltpu
from jax.experimental.pallas import tpu_sc as plsc

_K = 81
_N = 4096
_NV = _N // 16
_INT_MIN = -2147483648
_W = 4
_WSZ = _W * _N
_NW = 32
_R_BOT = _N - _K + 1
_SC_ROWS = 2560
_BR = 256


def _row_select(xw, ow, hist, candk, candi, roff, lane, lane32, ones16,
                zero16):
    for i in range(32):
        hist[pl.ds(i * 16, 16)] = zero16

    @plsc.parallel_loop(0, _NV, unroll=8)
    def _(j):
        b = lax.bitcast_convert_type(xw[pl.ds(roff + j * 16, 16)], jnp.int32)
        u = b ^ ((b >> 31) | jnp.int32(_INT_MIN))
        digit = lax.shift_right_logical(u, 27)
        plsc.addupdate_scatter(hist, [lane32 + digit], ones16)

    acc0 = zero16
    acc1 = zero16
    for l in range(16):
        acc0 = acc0 + hist[pl.ds(l * 32, 16)]
        acc1 = acc1 + hist[pl.ds(l * 32 + 16, 16)]
    p0 = plsc.cumsum(acc0)
    p1 = plsc.cumsum(acc1) + p0[15]
    m0 = p0 < _R_BOT
    m1 = p1 < _R_BOT
    bstar = (plsc.all_reduce_population_count(m0)[0]
             + plsc.all_reduce_population_count(m1)[0])
    pb_low = jnp.sum(jnp.where(m0, acc0, 0)) + jnp.sum(jnp.where(m1, acc1, 0))
    inb0 = jnp.logical_and(p0 >= _R_BOT, (p0 - acc0) < _R_BOT)
    inb1 = jnp.logical_and(p1 >= _R_BOT, (p1 - acc1) < _R_BOT)
    n_c = jnp.sum(jnp.where(inb0, acc0, 0)) + jnp.sum(jnp.where(inb1, acc1, 0))
    c_above = jnp.int32(_N) - pb_low - n_c
    m_rank = jnp.int32(_K) - c_above

    @plsc.parallel_loop(0, _NV, unroll=4, carry=zero16)
    def cnt(j, c):
        v = xw[pl.ds(roff + j * 16, 16)]
        b = lax.bitcast_convert_type(v, jnp.int32)
        u = b ^ ((b >> 31) | jnp.int32(_INT_MIN))
        digit = lax.shift_right_logical(u, 27)
        ow[pl.ds(roff + j * 16, 16)] = jnp.where(digit > bstar, v,
                                                 jnp.float32(0.0))
        inb = digit == bstar
        slot = lax.shift_left(c, 4) | lane
        plsc.store_scatter(candk, [slot], u ^ jnp.int32(_INT_MIN), mask=inb)
        plsc.store_scatter(candi, [slot], j * 16 + lane, mask=inb)
        return c + jnp.where(inb, 1, 0)

    mx = jnp.max(cnt)

    t = zero16 + ((bstar << 27) ^ jnp.int32(_INT_MIN))
    for shift in range(26, -1, -1):
        cand_t = t + jnp.int32(1 << shift)

        @plsc.parallel_loop(0, mx, carry=zero16)
        def acc_ge(s, acc):
            kv = candk[pl.ds(s * 16, 16)]
            ge = jnp.logical_and(kv >= cand_t, cnt > s)
            return acc + plsc.all_reduce_population_count(ge)

        t = jnp.where(acc_ge >= m_rank, cand_t, t)

    @plsc.parallel_loop(0, mx)
    def _(s):
        kv = candk[pl.ds(s * 16, 16)]
        iv = candi[pl.ds(s * 16, 16)]
        ge = jnp.logical_and(kv >= t, cnt > s)
        vf = lax.bitcast_convert_type(
            kv ^ ((kv >> 31) & jnp.int32(0x7FFFFFFF)), jnp.float32)
        plsc.store_scatter(ow, [roff + iv], vf, mask=ge)


def _sc_wta(rpw, row_off, x_hbm, out_hbm, xw0, xw1, ow0, ow1, hist, candk,
            candi, si0, si1, so0, so1):
    cid = lax.axis_index("c")
    sid = lax.axis_index("s")
    wid = sid * 2 + cid
    lane = lax.iota(jnp.int32, 16)
    lane32 = lane * 32
    ones16 = jnp.ones((16,), jnp.int32)
    zero16 = jnp.zeros((16,), jnp.int32)
    xwb = [xw0, xw1]
    owb = [ow0, ow1]
    sin = [si0, si1]
    sout = [so0, so1]
    nwin = rpw // _W
    base_out = wid * rpw * _N
    base_in = (row_off + wid * rpw) * _N

    for b in range(2):
        pltpu.make_async_copy(x_hbm.at[pl.ds(base_in + b * _WSZ, _WSZ)],
                              xwb[b], sin[b]).start()

    def pair(g, c):
        for b in range(2):
            w = g * 2 + b
            gin = base_in + w * _WSZ
            gout = base_out + w * _WSZ
            pltpu.make_async_copy(x_hbm.at[pl.ds(gin, _WSZ)], xwb[b],
                                  sin[b]).wait()

            @pl.when(g > 0)
            def _():
                pltpu.make_async_copy(owb[b],
                                      out_hbm.at[pl.ds(gout - 2 * _WSZ,
                                                       _WSZ)],
                                      sout[b]).wait()

            def row(r, c2):
                _row_select(xwb[b], owb[b], hist, candk, candi, r * _N,
                            lane, lane32, ones16, zero16)
                return c2

            lax.fori_loop(0, _W, row, jnp.int32(0))
            pltpu.make_async_copy(owb[b], out_hbm.at[pl.ds(gout, _WSZ)],
                                  sout[b]).start()

            @pl.when(w + 2 < nwin)
            def _():
                pltpu.make_async_copy(x_hbm.at[pl.ds(gin + 2 * _WSZ, _WSZ)],
                                      xwb[b], sin[b]).start()
        return c

    lax.fori_loop(0, nwin // 2, pair, jnp.int32(0))

    for b in range(2):
        pltpu.make_async_copy(owb[b], out_hbm.at[pl.ds(base_out, _WSZ)],
                              sout[b]).wait()


def _wta_tc_body(x_ref, o_ref):
    xb = x_ref[...]
    b = lax.bitcast_convert_type(xb, jnp.int32)
    skey = b ^ ((b >> 31) & jnp.int32(0x7FFFFFFF))
    rows = xb.shape[0]
    t = jnp.full((rows, 1), jnp.int32(_INT_MIN))
    steps = [jnp.int32(_INT_MIN)] + [jnp.int32(1 << s)
                                     for s in range(30, -1, -1)]
    for step in steps:
        cand = t + step
        cnt = jnp.sum((skey >= cand).astype(jnp.int32), axis=1, keepdims=True)
        t = jnp.where(cnt >= _K, cand, t)
    o_ref[...] = jnp.where(skey >= t, xb, jnp.float32(0.0))


def kernel(x):
    B, S, N = x.shape
    rows = B * S
    xf2 = x.reshape(rows, N)
    rows_sc = _SC_ROWS if rows % 256 == 0 and rows > _SC_ROWS else rows
    rows_tc = rows - rows_sc

    out_tc = None
    if rows_tc:
        out_tc = pl.pallas_call(
            _wta_tc_body,
            grid=(rows_tc // _BR,),
            in_specs=[pl.BlockSpec((_BR, N), lambda i: (i, 0))],
            out_specs=pl.BlockSpec((_BR, N), lambda i: (i, 0)),
            out_shape=jax.ShapeDtypeStruct((rows, N), jnp.float32),
            compiler_params=pltpu.CompilerParams(
                dimension_semantics=("parallel",),
            ),
        )(xf2)

    rpw = rows_sc // _NW
    mesh = plsc.VectorSubcoreMesh(core_axis_name="c", subcore_axis_name="s")
    run = functools.partial(
        pl.kernel,
        mesh=mesh,
        out_type=jax.ShapeDtypeStruct((rows_sc * N,), jnp.float32),
        scratch_types=[
            pltpu.VMEM((_WSZ,), jnp.float32),
            pltpu.VMEM((_WSZ,), jnp.float32),
            pltpu.VMEM((_WSZ,), jnp.float32),
            pltpu.VMEM((_WSZ,), jnp.float32),
            pltpu.VMEM((512,), jnp.int32),
            pltpu.VMEM((_N + 16,), jnp.int32),
            pltpu.VMEM((_N + 16,), jnp.int32),
            pltpu.SemaphoreType.DMA,
            pltpu.SemaphoreType.DMA,
            pltpu.SemaphoreType.DMA,
            pltpu.SemaphoreType.DMA,
        ],
        compiler_params=pltpu.CompilerParams(needs_layout_passes=False),
    )(functools.partial(_sc_wta, rpw, 0))
    out_sc = run(xf2[rows_tc:].reshape(-1)).reshape(rows_sc, N)

    if out_tc is None:
        return out_sc.reshape(B, S, N)

    def _patch_body(full_ref, sc_ref, o_ref):
        o_ref[...] = sc_ref[...]

    blk_off = rows_tc // _BR
    out = pl.pallas_call(
        _patch_body,
        grid=(rows_sc // _BR,),
        in_specs=[
            pl.BlockSpec((_BR, N), lambda i: (i + blk_off, 0)),
            pl.BlockSpec((_BR, N), lambda i: (i, 0)),
        ],
        out_specs=pl.BlockSpec((_BR, N), lambda i: (i + blk_off, 0)),
        out_shape=jax.ShapeDtypeStruct((rows, N), jnp.float32),
        input_output_aliases={0: 0},
        compiler_params=pltpu.CompilerParams(
            dimension_semantics=("parallel",),
        ),
    )(out_tc, out_sc)
    return out.reshape(B, S, N)
